# baseline (speedup 1.0000x reference)
import functools

import jax
import jax.numpy as jnp
from jax.experimental import pallas as pl
from jax.experimental.pallas import tpu as pltpu

BK = 40


def _body(xt_ref, et_ref, o_ref, e_ref, *, nblk):
    i = pl.program_id(0)

    @pl.when(i == 0)
    def _():
        e_ref[...] = et_ref[...].T

    x = xt_ref[...].astype(jnp.float32)
    e_slab = e_ref[pl.ds(i * BK, BK), :]
    prod_t = jax.lax.dot_general(
        e_slab, x, (((0,), (0,)), ((), ())),
        preferred_element_type=jnp.float32,
    )
    shifted = prod_t[:, 1:]

    @pl.when(i == 0)
    def _():
        o_ref[...] = shifted

    @pl.when(i > 0)
    def _():
        o_ref[...] += shifted


def kernel(inputs, embeddings):
    M, K = inputs.shape
    _, N = embeddings.shape
    xt = inputs.T
    et = embeddings.T
    nblk = K // BK
    out_t = pl.pallas_call(
        functools.partial(_body, nblk=nblk),
        grid=(nblk,),
        in_specs=[
            pl.BlockSpec((BK, M), lambda i: (i, 0)),
            pl.BlockSpec((N, K), lambda i: (0, 0)),
        ],
        out_specs=pl.BlockSpec((N, M - 1), lambda i: (0, 0)),
        out_shape=jax.ShapeDtypeStruct((N, M - 1), jnp.float32),
        scratch_shapes=[
            pltpu.VMEM((K, N), jnp.float32),
        ],
    )(xt, et)
    return out_t.T


# speedup vs baseline: 1.0380x; 1.0380x over previous
import jax
import jax.numpy as jnp
from jax.experimental import pallas as pl
from jax.experimental.pallas import tpu as pltpu

BN = 2048
BH = 1024


def _body(xt_ref, et_ref, o_ref, e_ref, prev_ref):
    i = pl.program_id(0)
    h = pl.program_id(1)

    @pl.when((i == 0) & (h == 0))
    def _():
        e_ref[...] = et_ref[...].T

    off = (1 - h) * BH
    x = xt_ref[:, pl.ds(off, BH)].astype(jnp.float32)
    prod_t = jax.lax.dot_general(
        e_ref[...], x, (((0,), (0,)), ((), ())),
        preferred_element_type=jnp.float32,
    )
    carry = prev_ref[...]
    o_ref[...] = jnp.concatenate([prod_t[:, 1:], carry], axis=1)
    prev_ref[...] = prod_t[:, 0:1]


def kernel(inputs, embeddings):
    M, K = inputs.shape
    _, N = embeddings.shape
    xt = inputs.T
    et = embeddings.T
    nblk = M // BN
    out_t = pl.pallas_call(
        _body,
        grid=(nblk, BN // BH),
        in_specs=[
            pl.BlockSpec((K, BN), lambda i, h, n=nblk: (0, n - 1 - i)),
            pl.BlockSpec((N, K), lambda i, h: (0, 0)),
        ],
        out_specs=pl.BlockSpec(
            (N, BH), lambda i, h, n=nblk: (0, (n - 1 - i) * (BN // BH) + 1 - h)
        ),
        out_shape=jax.ShapeDtypeStruct((N, M - 1), jnp.float32),
        scratch_shapes=[
            pltpu.VMEM((K, N), jnp.float32),
            pltpu.VMEM((N, 1), jnp.float32),
        ],
    )(xt, et)
    return out_t.T


# speedup vs baseline: 1.5879x; 1.5298x over previous
import jax
import jax.numpy as jnp
from jax.experimental import pallas as pl
from jax.experimental.pallas import tpu as pltpu


def _body(xt_ref, et_ref, o_ref, e_ref, prev_ref):
    i = pl.program_id(0)

    @pl.when(i == 0)
    def _():
        e_ref[...] = et_ref[...].T

    x = xt_ref[...].astype(jnp.float32)
    prod_t = jax.lax.dot_general(
        e_ref[...], x, (((0,), (0,)), ((), ())),
        preferred_element_type=jnp.float32,
    )
    carry = prev_ref[...]
    o_ref[...] = jnp.concatenate([prod_t[:, 1:], carry], axis=1)
    prev_ref[...] = prod_t[:, 0:1]


def kernel(inputs, embeddings):
    M, K = inputs.shape
    _, N = embeddings.shape
    xt = inputs.T
    et = embeddings.T
    BN = 2048
    nblk = M // BN
    out_t = pl.pallas_call(
        _body,
        grid=(nblk,),
        in_specs=[
            pl.BlockSpec((K, BN), lambda i, n=nblk: (0, n - 1 - i)),
            pl.BlockSpec((N, K), lambda i: (0, 0)),
        ],
        out_specs=pl.BlockSpec((N, BN), lambda i, n=nblk: (0, n - 1 - i)),
        out_shape=jax.ShapeDtypeStruct((N, M - 1), jnp.float32),
        scratch_shapes=[
            pltpu.VMEM((K, N), jnp.float32),
            pltpu.VMEM((N, 1), jnp.float32),
        ],
    )(xt, et)
    return out_t.T
